# baseline (speedup 1.0000x reference)
import functools

import jax
import jax.numpy as jnp
from jax import lax
from jax.experimental import pallas as pl
from jax.experimental.pallas import tpu as pltpu
from jax.experimental.pallas import tpu_sc as plsc

N = 10000
D_IN = 128
D_HID = 128
D_OUT = 64

NC = 2
NS = 16
L = 16

NP = 10240
SEG = NP // NS
EP = 335872
EB = EP // NS
BB = 1312
NBB = EB // BB
RB = 128
NBLK = EB // RB

RBLK = 400
GRID = N // RBLK


def _tc_layer_kernel(x_ref, w_ref, as_ref, ad_ref, h_ref, asn_ref, adn_ref):
    h = jnp.dot(x_ref[...], w_ref[...], preferred_element_type=jnp.float32)
    dh = h.shape[1] // 2
    h_ref[0] = h[:, :dh]
    h_ref[1] = h[:, dh:]
    asn_ref[...] = jnp.sum(h * as_ref[...], axis=1, keepdims=True)
    adn_ref[...] = jnp.sum(h * ad_ref[...], axis=1, keepdims=True)


def _tc_layer(x, W, a_s, a_d):
    din, dout = W.shape
    return pl.pallas_call(
        _tc_layer_kernel,
        grid=(GRID,),
        in_specs=[
            pl.BlockSpec((RBLK, din), lambda i: (i, 0)),
            pl.BlockSpec((din, dout), lambda i: (0, 0)),
            pl.BlockSpec((1, dout), lambda i: (0, 0)),
            pl.BlockSpec((1, dout), lambda i: (0, 0)),
        ],
        out_specs=[
            pl.BlockSpec((2, RBLK, dout // 2), lambda i: (0, i, 0)),
            pl.BlockSpec((RBLK, 1), lambda i: (i, 0)),
            pl.BlockSpec((RBLK, 1), lambda i: (i, 0)),
        ],
        out_shape=[
            jax.ShapeDtypeStruct((2, N, dout // 2), jnp.float32),
            jax.ShapeDtypeStruct((N, 1), jnp.float32),
            jax.ShapeDtypeStruct((N, 1), jnp.float32),
        ],
    )(x, W, a_s, a_d)


def _tc_mid_kernel(a0_ref, a1_ref, b_ref, w_ref, as_ref, ad_ref,
                   h_ref, asn_ref, adn_ref):
    z = jnp.concatenate([a0_ref[...], a1_ref[...]], axis=1)
    z = jax.nn.relu(z + b_ref[...])
    h = jnp.dot(z, w_ref[...], preferred_element_type=jnp.float32)
    dh = h.shape[1] // 2
    h_ref[0] = h[:, :dh]
    h_ref[1] = h[:, dh:]
    asn_ref[...] = jnp.sum(h * as_ref[...], axis=1, keepdims=True)
    adn_ref[...] = jnp.sum(h * ad_ref[...], axis=1, keepdims=True)


def _tc_mid(a0, a1, b, W, a_s, a_d):
    din, dout = W.shape
    return pl.pallas_call(
        _tc_mid_kernel,
        grid=(GRID,),
        in_specs=[
            pl.BlockSpec((RBLK, din // 2), lambda i: (i, 0)),
            pl.BlockSpec((RBLK, din // 2), lambda i: (i, 0)),
            pl.BlockSpec((1, din), lambda i: (0, 0)),
            pl.BlockSpec((din, dout), lambda i: (0, 0)),
            pl.BlockSpec((1, dout), lambda i: (0, 0)),
            pl.BlockSpec((1, dout), lambda i: (0, 0)),
        ],
        out_specs=[
            pl.BlockSpec((2, RBLK, dout // 2), lambda i: (0, i, 0)),
            pl.BlockSpec((RBLK, 1), lambda i: (i, 0)),
            pl.BlockSpec((RBLK, 1), lambda i: (i, 0)),
        ],
        out_shape=[
            jax.ShapeDtypeStruct((2, N, dout // 2), jnp.float32),
            jax.ShapeDtypeStruct((N, 1), jnp.float32),
            jax.ShapeDtypeStruct((N, 1), jnp.float32),
        ],
    )(a0, a1, b, W, a_s, a_d)


def _tc_final_kernel(a0_ref, a1_ref, b_ref, o_ref):
    z = jnp.concatenate([a0_ref[...], a1_ref[...]], axis=1)
    z = z + b_ref[...]
    m = jnp.max(z, axis=1, keepdims=True)
    zs = z - m
    o_ref[...] = zs - jnp.log(jnp.sum(jnp.exp(zs), axis=1, keepdims=True))


def _tc_final(a0, a1, b):
    dout = b.shape[1]
    return pl.pallas_call(
        _tc_final_kernel,
        grid=(GRID,),
        in_specs=[
            pl.BlockSpec((RBLK, dout // 2), lambda i: (i, 0)),
            pl.BlockSpec((RBLK, dout // 2), lambda i: (i, 0)),
            pl.BlockSpec((1, dout), lambda i: (0, 0)),
        ],
        out_specs=pl.BlockSpec((RBLK, dout), lambda i: (i, 0)),
        out_shape=jax.ShapeDtypeStruct((N, dout), jnp.float32),
    )(a0, a1, b)


@functools.lru_cache(maxsize=None)
def _make_sc_edge(D):
    DH = D // 2
    mesh = plsc.VectorSubcoreMesh(core_axis_name="c", subcore_axis_name="s",
                                  num_cores=NC, num_subcores=NS)

    @functools.partial(
        pl.kernel,
        out_type=jax.ShapeDtypeStruct((NC, NP, DH), jnp.float32),
        mesh=mesh,
        compiler_params=pltpu.CompilerParams(needs_layout_passes=False,
                                             use_tc_tiling_on_sc=False),
        scratch_types=[
            pltpu.VMEM((2, BB), jnp.int32),
            pltpu.VMEM((NP,), jnp.float32),
            pltpu.VMEM((NP,), jnp.float32),
            pltpu.VMEM((NP,), jnp.float32),
            pltpu.VMEM((NP,), jnp.float32),
            pltpu.VMEM((NS, SEG), jnp.float32),
            pltpu.VMEM((RB, DH), jnp.float32),
            pltpu.VMEM((RB, DH), jnp.float32),
            pltpu.VMEM((RB, DH), jnp.float32),
            pltpu.VMEM((RB, DH), jnp.float32),
            pltpu.VMEM((RB,), jnp.float32),
            pltpu.VMEM((4, 2, RB), jnp.int32),
            pltpu.VMEM((4, RB), jnp.int32),
            pltpu.VMEM((BB,), jnp.float32),
            pltpu.VMEM((4, RB), jnp.float32),
            pltpu.VMEM_SHARED((NP, DH), jnp.float32),
            pltpu.HBM((NC, NS, NP), jnp.float32),
            pltpu.HBM((NC, NP), jnp.float32),
            pltpu.HBM((NC, EP), jnp.float32),
            pltpu.SemaphoreType.DMA,
            pltpu.SemaphoreType.DMA,
            pltpu.SemaphoreType.DMA,
            pltpu.SemaphoreType.DMA,
            pltpu.SemaphoreType.DMA,
            pltpu.SemaphoreType.DMA,
            pltpu.SemaphoreType.DMA,
            pltpu.SemaphoreType.DMA,
            pltpu.SemaphoreType.DMA,
            pltpu.SemaphoreType.DMA,
            pltpu.SemaphoreType.DMA,
            pltpu.SemaphoreType.DMA,
            pltpu.SemaphoreType.DMA,
            pltpu.SemaphoreType.DMA,
            pltpu.SemaphoreType.DMA,
            pltpu.SemaphoreType.DMA,
        ],
    )
    def sc_edge(sdbb_hbm, sdrb_hbm, asn_hbm, adn_hbm, htab_hbm, out_hbm,
                sdb, as_t, ad_t, m_t, s_t, red_t, rows0, rows1, rows2, rows_3,
                alpha_b, sd3, sidx3, e_blk, ex3, acc_sh, stage, full_h,
                ex_hbm, g0, g1, g2, g3, s0, s1, s2, s3,
                i0, i1, i2, i3, x0, x1, x2, x3):
        cid = lax.axis_index("c")
        sid = lax.axis_index("s")
        rows3 = (rows0, rows1, rows2, rows_3)
        gsem = (g0, g1, g2, g3)
        ssem = (s0, s1, s2, s3)
        isem = (i0, i1, i2, i3)
        xsem = (x0, x1, x2, x3)

        pltpu.sync_copy(asn_hbm, as_t)
        pltpu.sync_copy(adn_hbm, ad_t)

        neg = jnp.full((L,), -3e38, jnp.float32)

        def init_m(i, _):
            m_t[pl.ds(i * L, L)] = neg
            return 0

        lax.fori_loop(0, NP // L, init_m, 0, unroll=8)

        def zrows(i, _):
            for v in range(DH // L):
                rows0[i, pl.ds(v * L, L)] = jnp.zeros((L,), jnp.float32)
            return 0

        lax.fori_loop(0, RB, zrows, 0)

        for j in range(SEG // RB):
            pltpu.sync_copy(rows0, acc_sh.at[pl.ds(sid * SEG + j * RB, RB)])

        def edge_vals(s16, d16):
            av = plsc.load_gather(as_t, [s16])
            dv = plsc.load_gather(ad_t, [d16])
            e = av + dv
            return jnp.where(e > 0, e, 0.2 * e)

        def m_blk(bi, _):
            pltpu.sync_copy(sdbb_hbm.at[sid * NBB + bi], sdb)

            def inner(i, _):
                s16 = sdb[0, pl.ds(i * L, L)]
                d16 = sdb[1, pl.ds(i * L, L)]
                e = edge_vals(s16, d16)
                e_blk[pl.ds(i * L, L)] = e
                cur = plsc.load_gather(m_t, [d16])
                mask0 = e > cur

                def cond(m):
                    return plsc.all_reduce_population_count(m)[0] > 0

                def body(m):
                    plsc.store_scatter(m_t, [d16], e, mask=m)
                    cur2 = plsc.load_gather(m_t, [d16])
                    return jnp.logical_and(m, e > cur2)

                lax.while_loop(cond, body, mask0)
                return 0

            lax.fori_loop(0, BB // L, inner, 0, unroll=4)
            pltpu.sync_copy(
                e_blk, ex_hbm.at[cid].at[pl.ds(sid * EB + bi * BB, BB)])
            return 0

        lax.fori_loop(0, NBB, m_blk, 0)

        def reduce_tiles(loc_ref, is_max):
            pltpu.sync_copy(loc_ref, stage.at[cid].at[sid])
            plsc.subcore_barrier()
            pltpu.sync_copy(stage.at[cid].at[:, pl.ds(sid * SEG, SEG)], red_t)

            def red_one(v, _):
                r = red_t[0, pl.ds(v * L, L)]
                for t in range(1, NS):
                    x = red_t[t, pl.ds(v * L, L)]
                    r = jnp.maximum(r, x) if is_max else r + x
                loc_ref[pl.ds(sid * SEG + v * L, L)] = r
                return 0

            lax.fori_loop(0, SEG // L, red_one, 0, unroll=2)
            pltpu.sync_copy(loc_ref.at[pl.ds(sid * SEG, SEG)],
                            full_h.at[cid].at[pl.ds(sid * SEG, SEG)])
            plsc.subcore_barrier()
            pltpu.sync_copy(full_h.at[cid], loc_ref)
            plsc.subcore_barrier()

        reduce_tiles(m_t, True)

        def init_s(i, _):
            s_t[pl.ds(i * L, L)] = jnp.zeros((L,), jnp.float32)
            return 0

        lax.fori_loop(0, NP // L, init_s, 0, unroll=8)

        def s_blk(bi, _):
            pltpu.sync_copy(sdbb_hbm.at[sid * NBB + bi], sdb)
            pltpu.sync_copy(
                ex_hbm.at[cid].at[pl.ds(sid * EB + bi * BB, BB)], e_blk)

            def inner(i, _):
                d16 = sdb[1, pl.ds(i * L, L)]
                e = e_blk[pl.ds(i * L, L)]
                mg = plsc.load_gather(m_t, [d16])
                ex = jnp.exp(e - mg)
                e_blk[pl.ds(i * L, L)] = ex
                plsc.addupdate_scatter(s_t, [d16], ex)
                return 0

            lax.fori_loop(0, BB // L, inner, 0, unroll=4)
            pltpu.sync_copy(
                e_blk, ex_hbm.at[cid].at[pl.ds(sid * EB + bi * BB, BB)])
            return 0

        lax.fori_loop(0, NBB, s_blk, 0)

        reduce_tiles(s_t, False)

        def issue_gather(p):
            pltpu.async_copy(
                htab_hbm.at[cid].at[sd3.at[p, 0]], rows3[p], gsem[p])

        def wait_gather(p):
            pltpu.make_async_copy(
                htab_hbm.at[cid].at[sd3.at[p, 0]], rows3[p], gsem[p]).wait()

        def issue_scatter(p):
            pltpu.async_copy(
                rows3[p], acc_sh.at[sidx3.at[p]], ssem[p], add=True)

        def wait_scatter(p):
            pltpu.make_async_copy(
                rows3[p], acc_sh.at[sidx3.at[p]], ssem[p]).wait()

        def load_idx(p, bg):
            pltpu.async_copy(sdrb_hbm.at[sid * NBLK + bg], sd3.at[p], isem[p])
            pltpu.async_copy(
                ex_hbm.at[cid].at[pl.ds(sid * EB + bg * RB, RB)],
                ex3.at[p], xsem[p])

        def wait_idx(p):
            pltpu.make_async_copy(
                sdrb_hbm.at[sid * NBLK], sd3.at[p], isem[p]).wait()

        def wait_ex(p):
            pltpu.make_async_copy(
                ex_hbm.at[cid].at[pl.ds(sid * EB, RB)],
                ex3.at[p], xsem[p]).wait()

        load_idx(0, 0)
        load_idx(1, 1)
        load_idx(2, 2)
        wait_idx(0)
        issue_gather(0)
        wait_idx(1)
        issue_gather(1)

        def compute_block(p):
            wait_ex(p)
            for j in range(RB // L):
                d16 = sd3[p, 1, pl.ds(j * L, L)]
                sidx3[p, pl.ds(j * L, L)] = d16
                sg = plsc.load_gather(s_t, [d16])
                ex = ex3[p, pl.ds(j * L, L)]
                alpha_b[pl.ds(j * L, L)] = ex / (sg + 1e-16)

            def scale_group(g, _):
                a16 = alpha_b[pl.ds(g * L, L)]
                for l in range(L):
                    a = a16[l]
                    r = g * L + l
                    for v in range(DH // L):
                        rows3[p][r, pl.ds(v * L, L)] = (
                            rows3[p][r, pl.ds(v * L, L)] * a)
                return 0

            lax.fori_loop(0, RB // L, scale_group, 0)

        def row_quad(t, _):
            for k in range(4):
                bg = t * 4 + k
                p = k
                p2 = (k + 2) % 4
                p3 = (k + 3) % 4

                @pl.when(bg >= 2)
                def _():
                    wait_scatter(p2)

                @pl.when(bg < NBLK - 2)
                def _():
                    wait_idx(p2)
                    issue_gather(p2)

                @pl.when(bg < NBLK - 3)
                def _():
                    load_idx(p3, bg + 3)

                wait_gather(p)
                compute_block(p)
                issue_scatter(p)
            return 0

        lax.fori_loop(0, NBLK // 4, row_quad, 0)

        wait_scatter(2)
        wait_scatter(3)

        plsc.subcore_barrier()
        pltpu.sync_copy(acc_sh.at[pl.ds(sid * SEG, SEG)],
                        out_hbm.at[cid].at[pl.ds(sid * SEG, SEG)])

    return sc_edge


def kernel(x, edge_index, new_edge_indexs, W1, a_s1, a_d1, b1,
           W2, a_s2, a_d2, b2):
    loops = jnp.arange(N, dtype=jnp.int32)
    pad = EP - (edge_index.shape[1] + N)
    src = jnp.concatenate([edge_index[0], loops,
                           jnp.zeros((pad,), jnp.int32)])
    dst = jnp.concatenate([edge_index[1], loops,
                           jnp.full((pad,), N, jnp.int32)])
    sd_bb = jnp.stack([src.reshape(-1, BB), dst.reshape(-1, BB)], axis=1)
    sd_rb = jnp.stack([src.reshape(-1, RB), dst.reshape(-1, RB)], axis=1)

    zpad = jnp.zeros((NP - N,), jnp.float32)

    h1, asn1, adn1 = _tc_layer(x, W1, a_s1.reshape(1, -1), a_d1.reshape(1, -1))
    asn1p = jnp.concatenate([asn1.reshape(-1), zpad])
    adn1p = jnp.concatenate([adn1.reshape(-1), zpad])
    acc1 = _make_sc_edge(D_HID)(sd_bb, sd_rb, asn1p, adn1p, h1)

    h2, asn2, adn2 = _tc_mid(acc1[0, :N], acc1[1, :N], b1.reshape(1, -1),
                             W2, a_s2.reshape(1, -1), a_d2.reshape(1, -1))
    asn2p = jnp.concatenate([asn2.reshape(-1), zpad])
    adn2p = jnp.concatenate([adn2.reshape(-1), zpad])
    acc2 = _make_sc_edge(D_OUT)(sd_bb, sd_rb, asn2p, adn2p, h2)

    return _tc_final(acc2[0, :N], acc2[1, :N], b2.reshape(1, -1))


# speedup vs baseline: 1.1427x; 1.1427x over previous
import functools

import jax
import jax.numpy as jnp
from jax import lax
from jax.experimental import pallas as pl
from jax.experimental.pallas import tpu as pltpu
from jax.experimental.pallas import tpu_sc as plsc

N = 10000
D_IN = 128
D_HID = 128
D_OUT = 64

NC = 2
NS = 16
L = 16

NP = 10240
SEG = NP // NS
EP = 331776
EB = EP // NS
BB = 2592
NBB = EB // BB
RB = 128
NBLK = EB // RB

RBLK = 400
GRID = N // RBLK


def _tc_layer_kernel(x_ref, w_ref, as_ref, ad_ref, h_ref, asn_ref, adn_ref):
    h = jnp.dot(x_ref[...], w_ref[...], preferred_element_type=jnp.float32)
    dh = h.shape[1] // 2
    h_ref[0] = h[:, :dh]
    h_ref[1] = h[:, dh:]
    asn_ref[...] = jnp.sum(h * as_ref[...], axis=1, keepdims=True)
    adn_ref[...] = jnp.sum(h * ad_ref[...], axis=1, keepdims=True)


def _tc_layer(x, W, a_s, a_d):
    din, dout = W.shape
    return pl.pallas_call(
        _tc_layer_kernel,
        grid=(GRID,),
        in_specs=[
            pl.BlockSpec((RBLK, din), lambda i: (i, 0)),
            pl.BlockSpec((din, dout), lambda i: (0, 0)),
            pl.BlockSpec((1, dout), lambda i: (0, 0)),
            pl.BlockSpec((1, dout), lambda i: (0, 0)),
        ],
        out_specs=[
            pl.BlockSpec((2, RBLK, dout // 2), lambda i: (0, i, 0)),
            pl.BlockSpec((RBLK, 1), lambda i: (i, 0)),
            pl.BlockSpec((RBLK, 1), lambda i: (i, 0)),
        ],
        out_shape=[
            jax.ShapeDtypeStruct((2, N, dout // 2), jnp.float32),
            jax.ShapeDtypeStruct((N, 1), jnp.float32),
            jax.ShapeDtypeStruct((N, 1), jnp.float32),
        ],
    )(x, W, a_s, a_d)


def _tc_mid_kernel(a0_ref, a1_ref, b_ref, w_ref, as_ref, ad_ref,
                   h_ref, asn_ref, adn_ref):
    z = jnp.concatenate([a0_ref[...], a1_ref[...]], axis=1)
    z = jax.nn.relu(z + b_ref[...])
    h = jnp.dot(z, w_ref[...], preferred_element_type=jnp.float32)
    dh = h.shape[1] // 2
    h_ref[0] = h[:, :dh]
    h_ref[1] = h[:, dh:]
    asn_ref[...] = jnp.sum(h * as_ref[...], axis=1, keepdims=True)
    adn_ref[...] = jnp.sum(h * ad_ref[...], axis=1, keepdims=True)


def _tc_mid(a0, a1, b, W, a_s, a_d):
    din, dout = W.shape
    return pl.pallas_call(
        _tc_mid_kernel,
        grid=(GRID,),
        in_specs=[
            pl.BlockSpec((RBLK, din // 2), lambda i: (i, 0)),
            pl.BlockSpec((RBLK, din // 2), lambda i: (i, 0)),
            pl.BlockSpec((1, din), lambda i: (0, 0)),
            pl.BlockSpec((din, dout), lambda i: (0, 0)),
            pl.BlockSpec((1, dout), lambda i: (0, 0)),
            pl.BlockSpec((1, dout), lambda i: (0, 0)),
        ],
        out_specs=[
            pl.BlockSpec((2, RBLK, dout // 2), lambda i: (0, i, 0)),
            pl.BlockSpec((RBLK, 1), lambda i: (i, 0)),
            pl.BlockSpec((RBLK, 1), lambda i: (i, 0)),
        ],
        out_shape=[
            jax.ShapeDtypeStruct((2, N, dout // 2), jnp.float32),
            jax.ShapeDtypeStruct((N, 1), jnp.float32),
            jax.ShapeDtypeStruct((N, 1), jnp.float32),
        ],
    )(a0, a1, b, W, a_s, a_d)


def _tc_final_kernel(a0_ref, a1_ref, b_ref, o_ref):
    z = jnp.concatenate([a0_ref[...], a1_ref[...]], axis=1)
    z = z + b_ref[...]
    m = jnp.max(z, axis=1, keepdims=True)
    zs = z - m
    o_ref[...] = zs - jnp.log(jnp.sum(jnp.exp(zs), axis=1, keepdims=True))


def _tc_final(a0, a1, b):
    dout = b.shape[1]
    return pl.pallas_call(
        _tc_final_kernel,
        grid=(GRID,),
        in_specs=[
            pl.BlockSpec((RBLK, dout // 2), lambda i: (i, 0)),
            pl.BlockSpec((RBLK, dout // 2), lambda i: (i, 0)),
            pl.BlockSpec((1, dout), lambda i: (0, 0)),
        ],
        out_specs=pl.BlockSpec((RBLK, dout), lambda i: (i, 0)),
        out_shape=jax.ShapeDtypeStruct((N, dout), jnp.float32),
    )(a0, a1, b)


@functools.lru_cache(maxsize=None)
def _make_sc_edge(D):
    DH = D // 2
    mesh = plsc.VectorSubcoreMesh(core_axis_name="c", subcore_axis_name="s",
                                  num_cores=NC, num_subcores=NS)

    @functools.partial(
        pl.kernel,
        out_type=jax.ShapeDtypeStruct((NC, NP, DH), jnp.float32),
        mesh=mesh,
        compiler_params=pltpu.CompilerParams(needs_layout_passes=False,
                                             use_tc_tiling_on_sc=False),
        scratch_types=[
            pltpu.VMEM((2, BB), jnp.int32),
            pltpu.VMEM((NP,), jnp.float32),
            pltpu.VMEM((NP,), jnp.float32),
            pltpu.VMEM((NP,), jnp.float32),
            pltpu.VMEM((NP,), jnp.float32),
            pltpu.VMEM((NS, SEG), jnp.float32),
            pltpu.VMEM((RB, DH), jnp.float32),
            pltpu.VMEM((RB, DH), jnp.float32),
            pltpu.VMEM((RB, DH), jnp.float32),
            pltpu.VMEM((RB,), jnp.float32),
            pltpu.VMEM((3, 2, RB), jnp.int32),
            pltpu.VMEM((3, RB), jnp.int32),
            pltpu.VMEM((BB,), jnp.float32),
            pltpu.VMEM((3, RB), jnp.float32),
            pltpu.VMEM_SHARED((NP, DH), jnp.float32),
            pltpu.HBM((NC, NS, NP), jnp.float32),
            pltpu.HBM((NC, NP), jnp.float32),
            pltpu.HBM((NC, EP), jnp.float32),
            pltpu.SemaphoreType.DMA,
            pltpu.SemaphoreType.DMA,
            pltpu.SemaphoreType.DMA,
            pltpu.SemaphoreType.DMA,
            pltpu.SemaphoreType.DMA,
            pltpu.SemaphoreType.DMA,
            pltpu.SemaphoreType.DMA,
            pltpu.SemaphoreType.DMA,
            pltpu.SemaphoreType.DMA,
            pltpu.SemaphoreType.DMA,
            pltpu.SemaphoreType.DMA,
            pltpu.SemaphoreType.DMA,
        ],
    )
    def sc_edge(sdbb_hbm, sdrb_hbm, asn_hbm, adn_hbm, htab_hbm, out_hbm,
                sdb, as_t, ad_t, m_t, s_t, red_t, rows0, rows1, rows2,
                alpha_b, sd3, sidx3, e_blk, ex3, acc_sh, stage, full_h,
                ex_hbm, g0, g1, g2, s0, s1, s2, i0, i1, i2, x0, x1, x2):
        cid = lax.axis_index("c")
        sid = lax.axis_index("s")
        rows3 = (rows0, rows1, rows2)
        gsem = (g0, g1, g2)
        ssem = (s0, s1, s2)
        isem = (i0, i1, i2)
        xsem = (x0, x1, x2)

        pltpu.sync_copy(asn_hbm, as_t)
        pltpu.sync_copy(adn_hbm, ad_t)

        neg = jnp.full((L,), -3e38, jnp.float32)

        def init_m(i, _):
            m_t[pl.ds(i * L, L)] = neg
            return 0

        lax.fori_loop(0, NP // L, init_m, 0, unroll=8)

        def zrows(i, _):
            for v in range(DH // L):
                rows0[i, pl.ds(v * L, L)] = jnp.zeros((L,), jnp.float32)
            return 0

        lax.fori_loop(0, RB, zrows, 0)

        for j in range(SEG // RB):
            pltpu.sync_copy(rows0, acc_sh.at[pl.ds(sid * SEG + j * RB, RB)])

        def edge_vals(s16, d16):
            av = plsc.load_gather(as_t, [s16])
            dv = plsc.load_gather(ad_t, [d16])
            e = av + dv
            return jnp.where(e > 0, e, 0.2 * e)

        def m_blk(bi, _):
            pltpu.sync_copy(sdbb_hbm.at[sid * NBB + bi], sdb)

            def inner(i, _):
                s16 = sdb[0, pl.ds(i * L, L)]
                d16 = sdb[1, pl.ds(i * L, L)]
                e = edge_vals(s16, d16)
                e_blk[pl.ds(i * L, L)] = e
                cur = plsc.load_gather(m_t, [d16])
                mask0 = e > cur

                def cond(m):
                    return plsc.all_reduce_population_count(m)[0] > 0

                def body(m):
                    plsc.store_scatter(m_t, [d16], e, mask=m)
                    cur2 = plsc.load_gather(m_t, [d16])
                    return jnp.logical_and(m, e > cur2)

                lax.while_loop(cond, body, mask0)
                return 0

            lax.fori_loop(0, BB // L, inner, 0, unroll=4)
            pltpu.sync_copy(
                e_blk, ex_hbm.at[cid].at[pl.ds(sid * EB + bi * BB, BB)])
            return 0

        lax.fori_loop(0, NBB, m_blk, 0)

        def reduce_tiles(loc_ref, is_max):
            pltpu.sync_copy(loc_ref, stage.at[cid].at[sid])
            plsc.subcore_barrier()
            pltpu.sync_copy(stage.at[cid].at[:, pl.ds(sid * SEG, SEG)], red_t)

            def red_one(v, _):
                r = red_t[0, pl.ds(v * L, L)]
                for t in range(1, NS):
                    x = red_t[t, pl.ds(v * L, L)]
                    r = jnp.maximum(r, x) if is_max else r + x
                loc_ref[pl.ds(sid * SEG + v * L, L)] = r
                return 0

            lax.fori_loop(0, SEG // L, red_one, 0, unroll=2)
            pltpu.sync_copy(loc_ref.at[pl.ds(sid * SEG, SEG)],
                            full_h.at[cid].at[pl.ds(sid * SEG, SEG)])
            plsc.subcore_barrier()
            pltpu.sync_copy(full_h.at[cid], loc_ref)
            plsc.subcore_barrier()

        reduce_tiles(m_t, True)

        def init_s(i, _):
            s_t[pl.ds(i * L, L)] = jnp.zeros((L,), jnp.float32)
            return 0

        lax.fori_loop(0, NP // L, init_s, 0, unroll=8)

        def s_blk(bi, _):
            pltpu.sync_copy(sdbb_hbm.at[sid * NBB + bi], sdb)
            pltpu.sync_copy(
                ex_hbm.at[cid].at[pl.ds(sid * EB + bi * BB, BB)], e_blk)

            def inner(i, _):
                d16 = sdb[1, pl.ds(i * L, L)]
                e = e_blk[pl.ds(i * L, L)]
                mg = plsc.load_gather(m_t, [d16])
                ex = jnp.exp(e - mg)
                e_blk[pl.ds(i * L, L)] = ex
                plsc.addupdate_scatter(s_t, [d16], ex)
                return 0

            lax.fori_loop(0, BB // L, inner, 0, unroll=4)
            pltpu.sync_copy(
                e_blk, ex_hbm.at[cid].at[pl.ds(sid * EB + bi * BB, BB)])
            return 0

        lax.fori_loop(0, NBB, s_blk, 0)

        reduce_tiles(s_t, False)

        def issue_gather(p):
            pltpu.async_copy(
                htab_hbm.at[cid].at[sd3.at[p, 0]], rows3[p], gsem[p])

        def wait_gather(p):
            pltpu.make_async_copy(
                htab_hbm.at[cid].at[sd3.at[p, 0]], rows3[p], gsem[p]).wait()

        def issue_scatter(p):
            pltpu.async_copy(
                rows3[p], acc_sh.at[sidx3.at[p]], ssem[p], add=True)

        def wait_scatter(p):
            pltpu.make_async_copy(
                rows3[p], acc_sh.at[sidx3.at[p]], ssem[p]).wait()

        def load_idx(p, bg):
            pltpu.async_copy(sdrb_hbm.at[sid * NBLK + bg], sd3.at[p], isem[p])
            pltpu.async_copy(
                ex_hbm.at[cid].at[pl.ds(sid * EB + bg * RB, RB)],
                ex3.at[p], xsem[p])

        def wait_idx(p):
            pltpu.make_async_copy(
                sdrb_hbm.at[sid * NBLK], sd3.at[p], isem[p]).wait()

        def wait_ex(p):
            pltpu.make_async_copy(
                ex_hbm.at[cid].at[pl.ds(sid * EB, RB)],
                ex3.at[p], xsem[p]).wait()

        load_idx(0, 0)
        load_idx(1, 1)
        wait_idx(0)
        issue_gather(0)

        def compute_block(p):
            wait_ex(p)
            for j in range(RB // L):
                d16 = sd3[p, 1, pl.ds(j * L, L)]
                sidx3[p, pl.ds(j * L, L)] = d16
                sg = plsc.load_gather(s_t, [d16])
                ex = ex3[p, pl.ds(j * L, L)]
                alpha_b[pl.ds(j * L, L)] = ex / (sg + 1e-16)

            def scale_group(g, _):
                a16 = alpha_b[pl.ds(g * L, L)]
                for l in range(L):
                    a = a16[l]
                    r = g * L + l
                    for v in range(DH // L):
                        rows3[p][r, pl.ds(v * L, L)] = (
                            rows3[p][r, pl.ds(v * L, L)] * a)
                return 0

            lax.fori_loop(0, RB // L, scale_group, 0)

        def row_triple(t, _):
            for k in range(3):
                bg = t * 3 + k
                p = k
                pn = (k + 1) % 3
                pp = (k + 2) % 3

                @pl.when(bg >= 2)
                def _():
                    wait_scatter(pn)

                @pl.when(bg < NBLK - 1)
                def _():
                    wait_idx(pn)
                    issue_gather(pn)

                @pl.when(bg < NBLK - 2)
                def _():
                    load_idx(pp, bg + 2)

                wait_gather(p)
                compute_block(p)
                issue_scatter(p)
            return 0

        lax.fori_loop(0, NBLK // 3, row_triple, 0)

        wait_scatter(1)
        wait_scatter(2)

        plsc.subcore_barrier()
        pltpu.sync_copy(acc_sh.at[pl.ds(sid * SEG, SEG)],
                        out_hbm.at[cid].at[pl.ds(sid * SEG, SEG)])

    return sc_edge


def kernel(x, edge_index, new_edge_indexs, W1, a_s1, a_d1, b1,
           W2, a_s2, a_d2, b2):
    loops = jnp.arange(N, dtype=jnp.int32)
    pad = EP - (edge_index.shape[1] + N)
    src = jnp.concatenate([edge_index[0], loops,
                           jnp.zeros((pad,), jnp.int32)])
    dst = jnp.concatenate([edge_index[1], loops,
                           jnp.full((pad,), N, jnp.int32)])
    sd_bb = jnp.stack([src.reshape(-1, BB), dst.reshape(-1, BB)], axis=1)
    sd_rb = jnp.stack([src.reshape(-1, RB), dst.reshape(-1, RB)], axis=1)

    zpad = jnp.zeros((NP - N,), jnp.float32)

    h1, asn1, adn1 = _tc_layer(x, W1, a_s1.reshape(1, -1), a_d1.reshape(1, -1))
    asn1p = jnp.concatenate([asn1.reshape(-1), zpad])
    adn1p = jnp.concatenate([adn1.reshape(-1), zpad])
    acc1 = _make_sc_edge(D_HID)(sd_bb, sd_rb, asn1p, adn1p, h1)

    h2, asn2, adn2 = _tc_mid(acc1[0, :N], acc1[1, :N], b1.reshape(1, -1),
                             W2, a_s2.reshape(1, -1), a_d2.reshape(1, -1))
    asn2p = jnp.concatenate([asn2.reshape(-1), zpad])
    adn2p = jnp.concatenate([adn2.reshape(-1), zpad])
    acc2 = _make_sc_edge(D_OUT)(sd_bb, sd_rb, asn2p, adn2p, h2)

    return _tc_final(acc2[0, :N], acc2[1, :N], b2.reshape(1, -1))


# speedup vs baseline: 1.5456x; 1.3527x over previous
import functools

import jax
import jax.numpy as jnp
from jax import lax
from jax.experimental import pallas as pl
from jax.experimental.pallas import tpu as pltpu
from jax.experimental.pallas import tpu_sc as plsc

N = 10000
D_IN = 128
D_HID = 128
D_OUT = 64

NC = 2
NS = 16
L = 16

NP = 10240
SEG = NP // NS
EP = 331776
EB = EP // NS
BB = 2592
NBB = EB // BB
RB = 128
NBLK = EB // RB

RBLK = 400
GRID = N // RBLK


def _tc_layer_kernel(x_ref, w_ref, as_ref, ad_ref, h_ref, asn_ref, adn_ref):
    h = jnp.dot(x_ref[...], w_ref[...], preferred_element_type=jnp.float32)
    dh = h.shape[1] // 2
    h_ref[0] = h[:, :dh]
    h_ref[1] = h[:, dh:]
    asn_ref[...] = jnp.sum(h * as_ref[...], axis=1, keepdims=True)
    adn_ref[...] = jnp.sum(h * ad_ref[...], axis=1, keepdims=True)


def _tc_layer(x, W, a_s, a_d):
    din, dout = W.shape
    return pl.pallas_call(
        _tc_layer_kernel,
        grid=(GRID,),
        in_specs=[
            pl.BlockSpec((RBLK, din), lambda i: (i, 0)),
            pl.BlockSpec((din, dout), lambda i: (0, 0)),
            pl.BlockSpec((1, dout), lambda i: (0, 0)),
            pl.BlockSpec((1, dout), lambda i: (0, 0)),
        ],
        out_specs=[
            pl.BlockSpec((2, RBLK, dout // 2), lambda i: (0, i, 0)),
            pl.BlockSpec((RBLK, 1), lambda i: (i, 0)),
            pl.BlockSpec((RBLK, 1), lambda i: (i, 0)),
        ],
        out_shape=[
            jax.ShapeDtypeStruct((2, N, dout // 2), jnp.float32),
            jax.ShapeDtypeStruct((N, 1), jnp.float32),
            jax.ShapeDtypeStruct((N, 1), jnp.float32),
        ],
    )(x, W, a_s, a_d)


def _tc_mid_kernel(a0_ref, a1_ref, b_ref, w_ref, as_ref, ad_ref,
                   h_ref, asn_ref, adn_ref):
    z = jnp.concatenate([a0_ref[...], a1_ref[...]], axis=1)
    z = jax.nn.relu(z + b_ref[...])
    h = jnp.dot(z, w_ref[...], preferred_element_type=jnp.float32)
    dh = h.shape[1] // 2
    h_ref[0] = h[:, :dh]
    h_ref[1] = h[:, dh:]
    asn_ref[...] = jnp.sum(h * as_ref[...], axis=1, keepdims=True)
    adn_ref[...] = jnp.sum(h * ad_ref[...], axis=1, keepdims=True)


def _tc_mid(a0, a1, b, W, a_s, a_d):
    din, dout = W.shape
    return pl.pallas_call(
        _tc_mid_kernel,
        grid=(GRID,),
        in_specs=[
            pl.BlockSpec((RBLK, din // 2), lambda i: (i, 0)),
            pl.BlockSpec((RBLK, din // 2), lambda i: (i, 0)),
            pl.BlockSpec((1, din), lambda i: (0, 0)),
            pl.BlockSpec((din, dout), lambda i: (0, 0)),
            pl.BlockSpec((1, dout), lambda i: (0, 0)),
            pl.BlockSpec((1, dout), lambda i: (0, 0)),
        ],
        out_specs=[
            pl.BlockSpec((2, RBLK, dout // 2), lambda i: (0, i, 0)),
            pl.BlockSpec((RBLK, 1), lambda i: (i, 0)),
            pl.BlockSpec((RBLK, 1), lambda i: (i, 0)),
        ],
        out_shape=[
            jax.ShapeDtypeStruct((2, N, dout // 2), jnp.float32),
            jax.ShapeDtypeStruct((N, 1), jnp.float32),
            jax.ShapeDtypeStruct((N, 1), jnp.float32),
        ],
    )(a0, a1, b, W, a_s, a_d)


def _tc_final_kernel(a0_ref, a1_ref, b_ref, o_ref):
    z = jnp.concatenate([a0_ref[...], a1_ref[...]], axis=1)
    z = z + b_ref[...]
    m = jnp.max(z, axis=1, keepdims=True)
    zs = z - m
    o_ref[...] = zs - jnp.log(jnp.sum(jnp.exp(zs), axis=1, keepdims=True))


def _tc_final(a0, a1, b):
    dout = b.shape[1]
    return pl.pallas_call(
        _tc_final_kernel,
        grid=(GRID,),
        in_specs=[
            pl.BlockSpec((RBLK, dout // 2), lambda i: (i, 0)),
            pl.BlockSpec((RBLK, dout // 2), lambda i: (i, 0)),
            pl.BlockSpec((1, dout), lambda i: (0, 0)),
        ],
        out_specs=pl.BlockSpec((RBLK, dout), lambda i: (i, 0)),
        out_shape=jax.ShapeDtypeStruct((N, dout), jnp.float32),
    )(a0, a1, b)


@functools.lru_cache(maxsize=None)
def _make_sc_edge(D):
    DH = D // 2
    mesh = plsc.VectorSubcoreMesh(core_axis_name="c", subcore_axis_name="s",
                                  num_cores=NC, num_subcores=NS)

    @functools.partial(
        pl.kernel,
        out_type=jax.ShapeDtypeStruct((NC, NP, DH), jnp.float32),
        mesh=mesh,
        compiler_params=pltpu.CompilerParams(needs_layout_passes=False,
                                             use_tc_tiling_on_sc=False),
        scratch_types=[
            pltpu.VMEM((2, BB), jnp.int32),
            pltpu.VMEM((NP,), jnp.float32),
            pltpu.VMEM((NP,), jnp.float32),
            pltpu.VMEM((NP,), jnp.float32),
            pltpu.VMEM((NP,), jnp.float32),
            pltpu.VMEM((NS, SEG), jnp.float32),
            pltpu.VMEM((RB, DH), jnp.float32),
            pltpu.VMEM((RB, DH), jnp.float32),
            pltpu.VMEM((RB, DH), jnp.float32),
            pltpu.VMEM((RB,), jnp.float32),
            pltpu.VMEM((3, 2, RB), jnp.int32),
            pltpu.VMEM((3, RB), jnp.int32),
            pltpu.VMEM((BB,), jnp.float32),
            pltpu.VMEM((3, RB), jnp.float32),
            pltpu.VMEM_SHARED((NP, DH), jnp.float32),
            pltpu.HBM((NC, NS, NP), jnp.float32),
            pltpu.HBM((NC, NP), jnp.float32),
            pltpu.HBM((NC, EP), jnp.float32),
            pltpu.SemaphoreType.DMA,
            pltpu.SemaphoreType.DMA,
            pltpu.SemaphoreType.DMA,
            pltpu.SemaphoreType.DMA,
            pltpu.SemaphoreType.DMA,
            pltpu.SemaphoreType.DMA,
            pltpu.SemaphoreType.DMA,
            pltpu.SemaphoreType.DMA,
            pltpu.SemaphoreType.DMA,
            pltpu.SemaphoreType.DMA,
            pltpu.SemaphoreType.DMA,
            pltpu.SemaphoreType.DMA,
        ],
    )
    def sc_edge(sdbb_hbm, sdrb_hbm, asn_hbm, adn_hbm, htab_hbm, out_hbm,
                sdb, as_t, ad_t, m_t, s_t, red_t, rows0, rows1, rows2,
                alpha_b, sd3, sidx3, e_blk, ex3, acc_sh, stage, full_h,
                ex_hbm, g0, g1, g2, s0, s1, s2, i0, i1, i2, x0, x1, x2):
        cid = lax.axis_index("c")
        sid = lax.axis_index("s")
        rows3 = (rows0, rows1, rows2)
        gsem = (g0, g1, g2)
        ssem = (s0, s1, s2)
        isem = (i0, i1, i2)
        xsem = (x0, x1, x2)

        pltpu.sync_copy(asn_hbm, as_t)
        pltpu.sync_copy(adn_hbm, ad_t)

        neg = jnp.full((L,), -3e38, jnp.float32)

        def init_m(i, _):
            m_t[pl.ds(i * L, L)] = neg
            return 0

        lax.fori_loop(0, NP // L, init_m, 0, unroll=8)

        def zrows(i, _):
            for v in range(DH // L):
                rows0[i, pl.ds(v * L, L)] = jnp.zeros((L,), jnp.float32)
            return 0

        lax.fori_loop(0, RB, zrows, 0)

        for j in range(SEG // RB):
            pltpu.sync_copy(rows0, acc_sh.at[pl.ds(sid * SEG + j * RB, RB)])

        def edge_vals(s16, d16):
            av = plsc.load_gather(as_t, [s16])
            dv = plsc.load_gather(ad_t, [d16])
            e = av + dv
            return jnp.where(e > 0, e, 0.2 * e)

        def m_blk(bi, _):
            pltpu.sync_copy(sdbb_hbm.at[sid * NBB + bi], sdb)

            def inner(i, _):
                s16 = sdb[0, pl.ds(i * L, L)]
                d16 = sdb[1, pl.ds(i * L, L)]
                e = edge_vals(s16, d16)
                e_blk[pl.ds(i * L, L)] = e
                cur = plsc.load_gather(m_t, [d16])
                mask0 = e > cur

                def cond(m):
                    return plsc.all_reduce_population_count(m)[0] > 0

                def body(m):
                    plsc.store_scatter(m_t, [d16], e, mask=m)
                    cur2 = plsc.load_gather(m_t, [d16])
                    return jnp.logical_and(m, e > cur2)

                lax.while_loop(cond, body, mask0)
                return 0

            lax.fori_loop(0, BB // L, inner, 0, unroll=4)
            pltpu.sync_copy(
                e_blk, ex_hbm.at[cid].at[pl.ds(sid * EB + bi * BB, BB)])
            return 0

        lax.fori_loop(0, NBB, m_blk, 0)

        def reduce_tiles(loc_ref, is_max):
            pltpu.sync_copy(loc_ref, stage.at[cid].at[sid])
            plsc.subcore_barrier()
            pltpu.sync_copy(stage.at[cid].at[:, pl.ds(sid * SEG, SEG)], red_t)

            def red_one(v, _):
                r = red_t[0, pl.ds(v * L, L)]
                for t in range(1, NS):
                    x = red_t[t, pl.ds(v * L, L)]
                    r = jnp.maximum(r, x) if is_max else r + x
                loc_ref[pl.ds(sid * SEG + v * L, L)] = r
                return 0

            lax.fori_loop(0, SEG // L, red_one, 0, unroll=2)
            pltpu.sync_copy(loc_ref.at[pl.ds(sid * SEG, SEG)],
                            full_h.at[cid].at[pl.ds(sid * SEG, SEG)])
            plsc.subcore_barrier()
            pltpu.sync_copy(full_h.at[cid], loc_ref)
            plsc.subcore_barrier()

        reduce_tiles(m_t, True)

        def init_s(i, _):
            s_t[pl.ds(i * L, L)] = jnp.zeros((L,), jnp.float32)
            return 0

        lax.fori_loop(0, NP // L, init_s, 0, unroll=8)

        def s_blk(bi, _):
            pltpu.sync_copy(sdbb_hbm.at[sid * NBB + bi], sdb)
            pltpu.sync_copy(
                ex_hbm.at[cid].at[pl.ds(sid * EB + bi * BB, BB)], e_blk)

            @plsc.parallel_loop(0, BB // L, unroll=4)
            def inner(i):
                d16 = sdb[1, pl.ds(i * L, L)]
                e = e_blk[pl.ds(i * L, L)]
                mg = plsc.load_gather(m_t, [d16])
                ex = jnp.exp(e - mg)
                e_blk[pl.ds(i * L, L)] = ex
                plsc.addupdate_scatter(s_t, [d16], ex)
            pltpu.sync_copy(
                e_blk, ex_hbm.at[cid].at[pl.ds(sid * EB + bi * BB, BB)])
            return 0

        lax.fori_loop(0, NBB, s_blk, 0)

        reduce_tiles(s_t, False)

        def issue_gather(p):
            pltpu.async_copy(
                htab_hbm.at[cid].at[sd3.at[p, 0]], rows3[p], gsem[p])

        def wait_gather(p):
            pltpu.make_async_copy(
                htab_hbm.at[cid].at[sd3.at[p, 0]], rows3[p], gsem[p]).wait()

        def issue_scatter(p):
            pltpu.async_copy(
                rows3[p], acc_sh.at[sidx3.at[p]], ssem[p], add=True)

        def wait_scatter(p):
            pltpu.make_async_copy(
                rows3[p], acc_sh.at[sidx3.at[p]], ssem[p]).wait()

        def load_idx(p, bg):
            pltpu.async_copy(sdrb_hbm.at[sid * NBLK + bg], sd3.at[p], isem[p])
            pltpu.async_copy(
                ex_hbm.at[cid].at[pl.ds(sid * EB + bg * RB, RB)],
                ex3.at[p], xsem[p])

        def wait_idx(p):
            pltpu.make_async_copy(
                sdrb_hbm.at[sid * NBLK], sd3.at[p], isem[p]).wait()

        def wait_ex(p):
            pltpu.make_async_copy(
                ex_hbm.at[cid].at[pl.ds(sid * EB, RB)],
                ex3.at[p], xsem[p]).wait()

        load_idx(0, 0)
        load_idx(1, 1)
        wait_idx(0)
        issue_gather(0)

        def compute_block(p):
            wait_ex(p)
            for j in range(RB // L):
                d16 = sd3[p, 1, pl.ds(j * L, L)]
                sidx3[p, pl.ds(j * L, L)] = d16
                sg = plsc.load_gather(s_t, [d16])
                ex = ex3[p, pl.ds(j * L, L)]
                alpha_b[pl.ds(j * L, L)] = ex / (sg + 1e-16)

            @plsc.parallel_loop(0, RB // L, unroll=2)
            def scale_group(g):
                a16 = alpha_b[pl.ds(g * L, L)]
                for l in range(L):
                    a = a16[l]
                    r = g * L + l
                    for v in range(DH // L):
                        rows3[p][r, pl.ds(v * L, L)] = (
                            rows3[p][r, pl.ds(v * L, L)] * a)

        def row_triple(t, _):
            for k in range(3):
                bg = t * 3 + k
                p = k
                pn = (k + 1) % 3
                pp = (k + 2) % 3

                @pl.when(bg >= 2)
                def _():
                    wait_scatter(pn)

                @pl.when(bg < NBLK - 1)
                def _():
                    wait_idx(pn)
                    issue_gather(pn)

                @pl.when(bg < NBLK - 2)
                def _():
                    load_idx(pp, bg + 2)

                wait_gather(p)
                compute_block(p)
                issue_scatter(p)
            return 0

        lax.fori_loop(0, NBLK // 3, row_triple, 0)

        wait_scatter(1)
        wait_scatter(2)

        plsc.subcore_barrier()
        pltpu.sync_copy(acc_sh.at[pl.ds(sid * SEG, SEG)],
                        out_hbm.at[cid].at[pl.ds(sid * SEG, SEG)])

    return sc_edge


def kernel(x, edge_index, new_edge_indexs, W1, a_s1, a_d1, b1,
           W2, a_s2, a_d2, b2):
    loops = jnp.arange(N, dtype=jnp.int32)
    pad = EP - (edge_index.shape[1] + N)
    src = jnp.concatenate([edge_index[0], loops,
                           jnp.zeros((pad,), jnp.int32)])
    dst = jnp.concatenate([edge_index[1], loops,
                           jnp.full((pad,), N, jnp.int32)])
    sd_bb = jnp.stack([src.reshape(-1, BB), dst.reshape(-1, BB)], axis=1)
    sd_rb = jnp.stack([src.reshape(-1, RB), dst.reshape(-1, RB)], axis=1)

    zpad = jnp.zeros((NP - N,), jnp.float32)

    h1, asn1, adn1 = _tc_layer(x, W1, a_s1.reshape(1, -1), a_d1.reshape(1, -1))
    asn1p = jnp.concatenate([asn1.reshape(-1), zpad])
    adn1p = jnp.concatenate([adn1.reshape(-1), zpad])
    acc1 = _make_sc_edge(D_HID)(sd_bb, sd_rb, asn1p, adn1p, h1)

    h2, asn2, adn2 = _tc_mid(acc1[0, :N], acc1[1, :N], b1.reshape(1, -1),
                             W2, a_s2.reshape(1, -1), a_d2.reshape(1, -1))
    asn2p = jnp.concatenate([asn2.reshape(-1), zpad])
    adn2p = jnp.concatenate([adn2.reshape(-1), zpad])
    acc2 = _make_sc_edge(D_OUT)(sd_bb, sd_rb, asn2p, adn2p, h2)

    return _tc_final(acc2[0, :N], acc2[1, :N], b2.reshape(1, -1))


# speedup vs baseline: 1.5524x; 1.0044x over previous
import functools

import jax
import jax.numpy as jnp
from jax import lax
from jax.experimental import pallas as pl
from jax.experimental.pallas import tpu as pltpu
from jax.experimental.pallas import tpu_sc as plsc

N = 10000
D_IN = 128
D_HID = 128
D_OUT = 64

NC = 2
NS = 16
L = 16

NP = 10240
SEG = NP // NS
EP = 331776
EB = EP // NS
BB = 2592
NBB = EB // BB
RB = 128
NBLK = EB // RB

RBLK = 400
GRID = N // RBLK


def _tc_layer_kernel(x_ref, w_ref, as_ref, ad_ref, h_ref, asn_ref, adn_ref):
    h = jnp.dot(x_ref[...], w_ref[...], preferred_element_type=jnp.float32)
    dh = h.shape[1] // 2
    h_ref[0] = h[:, :dh]
    h_ref[1] = h[:, dh:]
    asn_ref[...] = jnp.sum(h * as_ref[...], axis=1, keepdims=True)
    adn_ref[...] = jnp.sum(h * ad_ref[...], axis=1, keepdims=True)


def _tc_layer(x, W, a_s, a_d):
    din, dout = W.shape
    return pl.pallas_call(
        _tc_layer_kernel,
        grid=(GRID,),
        in_specs=[
            pl.BlockSpec((RBLK, din), lambda i: (i, 0)),
            pl.BlockSpec((din, dout), lambda i: (0, 0)),
            pl.BlockSpec((1, dout), lambda i: (0, 0)),
            pl.BlockSpec((1, dout), lambda i: (0, 0)),
        ],
        out_specs=[
            pl.BlockSpec((2, RBLK, dout // 2), lambda i: (0, i, 0)),
            pl.BlockSpec((RBLK, 1), lambda i: (i, 0)),
            pl.BlockSpec((RBLK, 1), lambda i: (i, 0)),
        ],
        out_shape=[
            jax.ShapeDtypeStruct((2, N, dout // 2), jnp.float32),
            jax.ShapeDtypeStruct((N, 1), jnp.float32),
            jax.ShapeDtypeStruct((N, 1), jnp.float32),
        ],
    )(x, W, a_s, a_d)


def _tc_mid_kernel(a0_ref, a1_ref, b_ref, w_ref, as_ref, ad_ref,
                   h_ref, asn_ref, adn_ref):
    z = jnp.concatenate([a0_ref[...], a1_ref[...]], axis=1)
    z = jax.nn.relu(z + b_ref[...])
    h = jnp.dot(z, w_ref[...], preferred_element_type=jnp.float32)
    dh = h.shape[1] // 2
    h_ref[0] = h[:, :dh]
    h_ref[1] = h[:, dh:]
    asn_ref[...] = jnp.sum(h * as_ref[...], axis=1, keepdims=True)
    adn_ref[...] = jnp.sum(h * ad_ref[...], axis=1, keepdims=True)


def _tc_mid(a0, a1, b, W, a_s, a_d):
    din, dout = W.shape
    return pl.pallas_call(
        _tc_mid_kernel,
        grid=(GRID,),
        in_specs=[
            pl.BlockSpec((RBLK, din // 2), lambda i: (i, 0)),
            pl.BlockSpec((RBLK, din // 2), lambda i: (i, 0)),
            pl.BlockSpec((1, din), lambda i: (0, 0)),
            pl.BlockSpec((din, dout), lambda i: (0, 0)),
            pl.BlockSpec((1, dout), lambda i: (0, 0)),
            pl.BlockSpec((1, dout), lambda i: (0, 0)),
        ],
        out_specs=[
            pl.BlockSpec((2, RBLK, dout // 2), lambda i: (0, i, 0)),
            pl.BlockSpec((RBLK, 1), lambda i: (i, 0)),
            pl.BlockSpec((RBLK, 1), lambda i: (i, 0)),
        ],
        out_shape=[
            jax.ShapeDtypeStruct((2, N, dout // 2), jnp.float32),
            jax.ShapeDtypeStruct((N, 1), jnp.float32),
            jax.ShapeDtypeStruct((N, 1), jnp.float32),
        ],
    )(a0, a1, b, W, a_s, a_d)


def _tc_final_kernel(a0_ref, a1_ref, b_ref, o_ref):
    z = jnp.concatenate([a0_ref[...], a1_ref[...]], axis=1)
    z = z + b_ref[...]
    m = jnp.max(z, axis=1, keepdims=True)
    zs = z - m
    o_ref[...] = zs - jnp.log(jnp.sum(jnp.exp(zs), axis=1, keepdims=True))


def _tc_final(a0, a1, b):
    dout = b.shape[1]
    return pl.pallas_call(
        _tc_final_kernel,
        grid=(GRID,),
        in_specs=[
            pl.BlockSpec((RBLK, dout // 2), lambda i: (i, 0)),
            pl.BlockSpec((RBLK, dout // 2), lambda i: (i, 0)),
            pl.BlockSpec((1, dout), lambda i: (0, 0)),
        ],
        out_specs=pl.BlockSpec((RBLK, dout), lambda i: (i, 0)),
        out_shape=jax.ShapeDtypeStruct((N, dout), jnp.float32),
    )(a0, a1, b)


@functools.lru_cache(maxsize=None)
def _make_sc_edge(D):
    DH = D // 2
    mesh = plsc.VectorSubcoreMesh(core_axis_name="c", subcore_axis_name="s",
                                  num_cores=NC, num_subcores=NS)

    @functools.partial(
        pl.kernel,
        out_type=jax.ShapeDtypeStruct((NC, NP, DH), jnp.float32),
        mesh=mesh,
        compiler_params=pltpu.CompilerParams(needs_layout_passes=False,
                                             use_tc_tiling_on_sc=False),
        scratch_types=[
            pltpu.VMEM((2, BB), jnp.int32),
            pltpu.VMEM((NP,), jnp.float32),
            pltpu.VMEM((NP,), jnp.float32),
            pltpu.VMEM((NP,), jnp.float32),
            pltpu.VMEM((NP,), jnp.float32),
            pltpu.VMEM((NS, SEG), jnp.float32),
            pltpu.VMEM((RB, DH), jnp.float32),
            pltpu.VMEM((RB, DH), jnp.float32),
            pltpu.VMEM((RB, DH), jnp.float32),
            pltpu.VMEM((RB,), jnp.float32),
            pltpu.VMEM((3, 2, RB), jnp.int32),
            pltpu.VMEM((3, RB), jnp.int32),
            pltpu.VMEM((BB,), jnp.float32),
            pltpu.VMEM((3, RB), jnp.float32),
            pltpu.VMEM_SHARED((NP, DH), jnp.float32),
            pltpu.HBM((NC, NS, NP), jnp.float32),
            pltpu.HBM((NC, NP), jnp.float32),
            pltpu.HBM((NC, EP), jnp.float32),
            pltpu.SemaphoreType.DMA,
            pltpu.SemaphoreType.DMA,
            pltpu.SemaphoreType.DMA,
            pltpu.SemaphoreType.DMA,
            pltpu.SemaphoreType.DMA,
            pltpu.SemaphoreType.DMA,
            pltpu.SemaphoreType.DMA,
            pltpu.SemaphoreType.DMA,
            pltpu.SemaphoreType.DMA,
            pltpu.SemaphoreType.DMA,
            pltpu.SemaphoreType.DMA,
            pltpu.SemaphoreType.DMA,
        ],
    )
    def sc_edge(sdbb_hbm, sdrb_hbm, asn_hbm, adn_hbm, htab_hbm, out_hbm,
                sdb, as_t, ad_t, m_t, s_t, red_t, rows0, rows1, rows2,
                alpha_b, sd3, sidx3, e_blk, ex3, acc_sh, stage, full_h,
                ex_hbm, g0, g1, g2, s0, s1, s2, i0, i1, i2, x0, x1, x2):
        cid = lax.axis_index("c")
        sid = lax.axis_index("s")
        rows3 = (rows0, rows1, rows2)
        gsem = (g0, g1, g2)
        ssem = (s0, s1, s2)
        isem = (i0, i1, i2)
        xsem = (x0, x1, x2)

        pltpu.sync_copy(asn_hbm, as_t)
        pltpu.sync_copy(adn_hbm, ad_t)

        neg = jnp.full((L,), -3e38, jnp.float32)

        @plsc.parallel_loop(0, NP // L, unroll=8)
        def init_m(i):
            m_t[pl.ds(i * L, L)] = neg

        @plsc.parallel_loop(0, RB, unroll=2)
        def zrows(i):
            for v in range(DH // L):
                rows0[i, pl.ds(v * L, L)] = jnp.zeros((L,), jnp.float32)

        for j in range(SEG // RB):
            pltpu.sync_copy(rows0, acc_sh.at[pl.ds(sid * SEG + j * RB, RB)])

        def edge_vals(s16, d16):
            av = plsc.load_gather(as_t, [s16])
            dv = plsc.load_gather(ad_t, [d16])
            e = av + dv
            return jnp.where(e > 0, e, 0.2 * e)

        def m_blk(bi, _):
            pltpu.sync_copy(sdbb_hbm.at[sid * NBB + bi], sdb)

            def inner(i, _):
                s16 = sdb[0, pl.ds(i * L, L)]
                d16 = sdb[1, pl.ds(i * L, L)]
                e = edge_vals(s16, d16)
                e_blk[pl.ds(i * L, L)] = e
                cur = plsc.load_gather(m_t, [d16])
                mask0 = e > cur

                def cond(m):
                    return plsc.all_reduce_population_count(m)[0] > 0

                def body(m):
                    plsc.store_scatter(m_t, [d16], e, mask=m)
                    cur2 = plsc.load_gather(m_t, [d16])
                    return jnp.logical_and(m, e > cur2)

                lax.while_loop(cond, body, mask0)
                return 0

            lax.fori_loop(0, BB // L, inner, 0, unroll=4)
            pltpu.sync_copy(
                e_blk, ex_hbm.at[cid].at[pl.ds(sid * EB + bi * BB, BB)])
            return 0

        lax.fori_loop(0, NBB, m_blk, 0)

        def reduce_tiles(loc_ref, is_max):
            pltpu.sync_copy(loc_ref, stage.at[cid].at[sid])
            plsc.subcore_barrier()
            pltpu.sync_copy(stage.at[cid].at[:, pl.ds(sid * SEG, SEG)], red_t)

            @plsc.parallel_loop(0, SEG // L, unroll=2)
            def red_one(v):
                r = red_t[0, pl.ds(v * L, L)]
                for t in range(1, NS):
                    x = red_t[t, pl.ds(v * L, L)]
                    r = jnp.maximum(r, x) if is_max else r + x
                loc_ref[pl.ds(sid * SEG + v * L, L)] = r
            pltpu.sync_copy(loc_ref.at[pl.ds(sid * SEG, SEG)],
                            full_h.at[cid].at[pl.ds(sid * SEG, SEG)])
            plsc.subcore_barrier()
            pltpu.sync_copy(full_h.at[cid], loc_ref)
            plsc.subcore_barrier()

        reduce_tiles(m_t, True)

        @plsc.parallel_loop(0, NP // L, unroll=8)
        def init_s(i):
            s_t[pl.ds(i * L, L)] = jnp.zeros((L,), jnp.float32)

        def s_blk(bi, _):
            pltpu.sync_copy(sdbb_hbm.at[sid * NBB + bi], sdb)
            pltpu.sync_copy(
                ex_hbm.at[cid].at[pl.ds(sid * EB + bi * BB, BB)], e_blk)

            @plsc.parallel_loop(0, BB // L, unroll=4)
            def inner(i):
                d16 = sdb[1, pl.ds(i * L, L)]
                e = e_blk[pl.ds(i * L, L)]
                mg = plsc.load_gather(m_t, [d16])
                ex = jnp.exp(e - mg)
                e_blk[pl.ds(i * L, L)] = ex
                plsc.addupdate_scatter(s_t, [d16], ex)
            pltpu.sync_copy(
                e_blk, ex_hbm.at[cid].at[pl.ds(sid * EB + bi * BB, BB)])
            return 0

        lax.fori_loop(0, NBB, s_blk, 0)

        reduce_tiles(s_t, False)

        def issue_gather(p):
            pltpu.async_copy(
                htab_hbm.at[cid].at[sd3.at[p, 0]], rows3[p], gsem[p])

        def wait_gather(p):
            pltpu.make_async_copy(
                htab_hbm.at[cid].at[sd3.at[p, 0]], rows3[p], gsem[p]).wait()

        def issue_scatter(p):
            pltpu.async_copy(
                rows3[p], acc_sh.at[sidx3.at[p]], ssem[p], add=True)

        def wait_scatter(p):
            pltpu.make_async_copy(
                rows3[p], acc_sh.at[sidx3.at[p]], ssem[p]).wait()

        def load_idx(p, bg):
            pltpu.async_copy(sdrb_hbm.at[sid * NBLK + bg], sd3.at[p], isem[p])
            pltpu.async_copy(
                ex_hbm.at[cid].at[pl.ds(sid * EB + bg * RB, RB)],
                ex3.at[p], xsem[p])

        def wait_idx(p):
            pltpu.make_async_copy(
                sdrb_hbm.at[sid * NBLK], sd3.at[p], isem[p]).wait()

        def wait_ex(p):
            pltpu.make_async_copy(
                ex_hbm.at[cid].at[pl.ds(sid * EB, RB)],
                ex3.at[p], xsem[p]).wait()

        load_idx(0, 0)
        load_idx(1, 1)
        wait_idx(0)
        issue_gather(0)

        def compute_block(p):
            wait_ex(p)
            for j in range(RB // L):
                d16 = sd3[p, 1, pl.ds(j * L, L)]
                sidx3[p, pl.ds(j * L, L)] = d16
                sg = plsc.load_gather(s_t, [d16])
                ex = ex3[p, pl.ds(j * L, L)]
                alpha_b[pl.ds(j * L, L)] = ex / (sg + 1e-16)

            @plsc.parallel_loop(0, RB // L, unroll=2)
            def scale_group(g):
                a16 = alpha_b[pl.ds(g * L, L)]
                for l in range(L):
                    a = a16[l]
                    r = g * L + l
                    for v in range(DH // L):
                        rows3[p][r, pl.ds(v * L, L)] = (
                            rows3[p][r, pl.ds(v * L, L)] * a)

        def row_triple(t, _):
            for k in range(3):
                bg = t * 3 + k
                p = k
                pn = (k + 1) % 3
                pp = (k + 2) % 3

                @pl.when(bg >= 2)
                def _():
                    wait_scatter(pn)

                @pl.when(bg < NBLK - 1)
                def _():
                    wait_idx(pn)
                    issue_gather(pn)

                @pl.when(bg < NBLK - 2)
                def _():
                    load_idx(pp, bg + 2)

                wait_gather(p)
                compute_block(p)
                issue_scatter(p)
            return 0

        lax.fori_loop(0, NBLK // 3, row_triple, 0)

        wait_scatter(1)
        wait_scatter(2)

        plsc.subcore_barrier()
        pltpu.sync_copy(acc_sh.at[pl.ds(sid * SEG, SEG)],
                        out_hbm.at[cid].at[pl.ds(sid * SEG, SEG)])

    return sc_edge


def kernel(x, edge_index, new_edge_indexs, W1, a_s1, a_d1, b1,
           W2, a_s2, a_d2, b2):
    loops = jnp.arange(N, dtype=jnp.int32)
    pad = EP - (edge_index.shape[1] + N)
    src = jnp.concatenate([edge_index[0], loops,
                           jnp.zeros((pad,), jnp.int32)])
    dst = jnp.concatenate([edge_index[1], loops,
                           jnp.full((pad,), N, jnp.int32)])
    sd_bb = jnp.stack([src.reshape(-1, BB), dst.reshape(-1, BB)], axis=1)
    sd_rb = jnp.stack([src.reshape(-1, RB), dst.reshape(-1, RB)], axis=1)

    zpad = jnp.zeros((NP - N,), jnp.float32)

    h1, asn1, adn1 = _tc_layer(x, W1, a_s1.reshape(1, -1), a_d1.reshape(1, -1))
    asn1p = jnp.concatenate([asn1.reshape(-1), zpad])
    adn1p = jnp.concatenate([adn1.reshape(-1), zpad])
    acc1 = _make_sc_edge(D_HID)(sd_bb, sd_rb, asn1p, adn1p, h1)

    h2, asn2, adn2 = _tc_mid(acc1[0, :N], acc1[1, :N], b1.reshape(1, -1),
                             W2, a_s2.reshape(1, -1), a_d2.reshape(1, -1))
    asn2p = jnp.concatenate([asn2.reshape(-1), zpad])
    adn2p = jnp.concatenate([adn2.reshape(-1), zpad])
    acc2 = _make_sc_edge(D_OUT)(sd_bb, sd_rb, asn2p, adn2p, h2)

    return _tc_final(acc2[0, :N], acc2[1, :N], b2.reshape(1, -1))
